# Initial kernel scaffold; baseline (speedup 1.0000x reference)
import jax
import jax.numpy as jnp
from jax.experimental import pallas as pl


def kernel(hidden_states, cos, sin, attention_mask, ln1_w, ln2_w, Wq, Wk, Wv, Wo, sinks, router_w, router_b, gate_up_proj, gate_up_bias, down_proj, down_bias):
    raise NotImplementedError("write your pallas kernel here")


# speedup vs baseline: 2.7319x; 2.7319x over previous
import jax
import jax.numpy as jnp
from jax.experimental import pallas as pl
from jax.experimental.pallas import tpu as pltpu

_call = pl.pallas_call

B, S, H = 1, 2048, 1024
NH, KVH, HD = 16, 4, 64
E, I = 8, 1024
EPS = 1e-06
ALPHA = 1.702
LIMIT = 7.0
SCALING = HD ** -0.5
BT = 256
NT = S // BT
QKV = NH * HD + 2 * KVH * HD
RH = HD // 2


def _qkv_body(x_ref, ln_ref, w_ref, o_ref):
    x = x_ref[...]
    var = jnp.mean(x * x, axis=-1, keepdims=True)
    xn = (x * jax.lax.rsqrt(var + EPS) * ln_ref[...]).astype(jnp.bfloat16)
    o_ref[...] = jnp.dot(xn, w_ref[...], preferred_element_type=jnp.float32)


def _rope(x, c, s):
    x1 = x[:, :RH]
    x2 = x[:, RH:]
    return jnp.concatenate([x1 * c - x2 * s, x2 * c + x1 * s], axis=1)


def _attn_body(q_ref, k_ref, v_ref, cq_ref, sq_ref, ck_ref, sk_ref,
               sink_ref, o_ref):
    h = pl.program_id(0)
    qr = _rope(q_ref[0], cq_ref[...], sq_ref[...]).astype(jnp.bfloat16)
    kr = _rope(k_ref[0], ck_ref[...], sk_ref[...]).astype(jnp.bfloat16)
    scores = jax.lax.dot_general(
        qr, kr, (((1,), (1,)), ((), ())),
        preferred_element_type=jnp.float32) * SCALING
    sel = jax.lax.broadcasted_iota(jnp.int32, (1, NH), 1) == h
    snk = jnp.sum(jnp.where(sel, sink_ref[...], 0.0), axis=1, keepdims=True)
    m = jnp.maximum(jnp.max(scores, axis=1, keepdims=True), snk)
    p = jnp.exp(scores - m)
    denom = jnp.sum(p, axis=1, keepdims=True) + jnp.exp(snk - m)
    probs = (p / denom).astype(jnp.bfloat16)
    o_ref[0] = jnp.dot(probs, v_ref[0].astype(jnp.bfloat16),
                       preferred_element_type=jnp.float32)


def _proj_router_body(ao_ref, wo_ref, res_ref, ln2_ref, rw_ref, rb_ref,
                      res2_ref, h2_ref, sc_ref):
    attn = jnp.dot(ao_ref[...].astype(jnp.bfloat16), wo_ref[...],
                   preferred_element_type=jnp.float32)
    res2 = res_ref[...] + attn
    res2_ref[...] = res2
    var = jnp.mean(res2 * res2, axis=-1, keepdims=True)
    h2 = res2 * jax.lax.rsqrt(var + EPS) * ln2_ref[...]
    h2_ref[...] = h2
    rl = jnp.dot(h2, rw_ref[...],
                 preferred_element_type=jnp.float32) + rb_ref[...]
    v1 = jnp.max(rl, axis=1, keepdims=True)
    rl_m = jnp.where(rl == v1, -jnp.inf, rl)
    v2 = jnp.max(rl_m, axis=1, keepdims=True)
    w1 = 1.0 / (1.0 + jnp.exp(v2 - v1))
    w2 = 1.0 - w1
    sc_ref[...] = jnp.where(rl == v1, w1, jnp.where(rl == v2, w2, 0.0))


def _gateup_body(x_ref, w_ref, b_ref, o_ref):
    gu = jnp.dot(x_ref[...], w_ref[0],
                 preferred_element_type=jnp.float32) + b_ref[0]
    gate = jnp.minimum(gu[:, :I], LIMIT)
    up = jnp.clip(gu[:, I:], -LIMIT, LIMIT)
    glu = gate * jax.nn.sigmoid(gate * ALPHA)
    o_ref[0] = ((up + 1.0) * glu).astype(jnp.bfloat16)


def _down_body(a_ref, w_ref, b_ref, sc_ref, res2_ref, o_ref, acc_ref):
    e = pl.program_id(1)
    y = jnp.dot(a_ref[0], w_ref[0],
                preferred_element_type=jnp.float32) + b_ref[0]
    sel = jax.lax.broadcasted_iota(jnp.int32, (1, E), 1) == e
    w_tok = jnp.sum(jnp.where(sel, sc_ref[...], 0.0), axis=1, keepdims=True)
    contrib = w_tok * y

    @pl.when(e == 0)
    def _():
        acc_ref[...] = res2_ref[...] + contrib

    @pl.when(e != 0)
    def _():
        acc_ref[...] += contrib

    @pl.when(e == E - 1)
    def _():
        o_ref[...] = acc_ref[...]


def kernel(hidden_states, cos, sin, attention_mask, ln1_w, ln2_w, Wq, Wk, Wv,
           Wo, sinks, router_w, router_b, gate_up_proj, gate_up_bias,
           down_proj, down_bias):
    f32 = jnp.float32
    bf16 = jnp.bfloat16
    x = hidden_states.reshape(S, H)
    wqkv = jnp.concatenate([Wq, Wk, Wv], axis=0).T.astype(bf16)

    qkv = _call(
        _qkv_body,
        grid=(NT,),
        in_specs=[
            pl.BlockSpec((BT, H), lambda i: (i, 0)),
            pl.BlockSpec((1, H), lambda i: (0, 0)),
            pl.BlockSpec((H, QKV), lambda i: (0, 0)),
        ],
        out_specs=pl.BlockSpec((BT, QKV), lambda i: (i, 0)),
        out_shape=jax.ShapeDtypeStruct((S, QKV), f32),
    )(x, ln1_w.reshape(1, H), wqkv)

    q = qkv[:, :NH * HD].reshape(S, NH, HD).transpose(1, 0, 2)
    k = qkv[:, NH * HD:NH * HD + KVH * HD].reshape(S, KVH, HD).transpose(1, 0, 2)
    v = qkv[:, NH * HD + KVH * HD:].reshape(S, KVH, HD).transpose(1, 0, 2)
    cosf = cos.reshape(S, RH)
    sinf = sin.reshape(S, RH)

    ao = _call(
        _attn_body,
        grid=(NH, NT),
        in_specs=[
            pl.BlockSpec((1, BT, HD), lambda h, t: (h, t, 0)),
            pl.BlockSpec((1, S, HD), lambda h, t: (h // 4, 0, 0)),
            pl.BlockSpec((1, S, HD), lambda h, t: (h // 4, 0, 0)),
            pl.BlockSpec((BT, RH), lambda h, t: (t, 0)),
            pl.BlockSpec((BT, RH), lambda h, t: (t, 0)),
            pl.BlockSpec((S, RH), lambda h, t: (0, 0)),
            pl.BlockSpec((S, RH), lambda h, t: (0, 0)),
            pl.BlockSpec((1, NH), lambda h, t: (0, 0)),
        ],
        out_specs=pl.BlockSpec((1, BT, HD), lambda h, t: (h, t, 0)),
        out_shape=jax.ShapeDtypeStruct((NH, S, HD), f32),
    )(q, k, v, cosf, sinf, cosf, sinf, sinks.reshape(1, NH))

    aof = ao.transpose(1, 0, 2).reshape(S, NH * HD)

    res2, h2, scores = _call(
        _proj_router_body,
        grid=(NT,),
        in_specs=[
            pl.BlockSpec((BT, NH * HD), lambda i: (i, 0)),
            pl.BlockSpec((NH * HD, H), lambda i: (0, 0)),
            pl.BlockSpec((BT, H), lambda i: (i, 0)),
            pl.BlockSpec((1, H), lambda i: (0, 0)),
            pl.BlockSpec((H, E), lambda i: (0, 0)),
            pl.BlockSpec((1, E), lambda i: (0, 0)),
        ],
        out_specs=[
            pl.BlockSpec((BT, H), lambda i: (i, 0)),
            pl.BlockSpec((BT, H), lambda i: (i, 0)),
            pl.BlockSpec((BT, E), lambda i: (i, 0)),
        ],
        out_shape=[
            jax.ShapeDtypeStruct((S, H), f32),
            jax.ShapeDtypeStruct((S, H), f32),
            jax.ShapeDtypeStruct((S, E), f32),
        ],
    )(aof, Wo.T.astype(bf16), x, ln2_w.reshape(1, H),
      router_w.T.astype(f32), router_b.reshape(1, E))

    h2b = h2.astype(bf16)
    gw = jnp.concatenate([gate_up_proj[..., ::2], gate_up_proj[..., 1::2]],
                         axis=-1).astype(bf16)
    gb = jnp.concatenate([gate_up_bias[:, ::2], gate_up_bias[:, 1::2]],
                         axis=-1).reshape(E, 1, 2 * I)

    act = _call(
        _gateup_body,
        grid=(E, NT),
        in_specs=[
            pl.BlockSpec((BT, H), lambda e, t: (t, 0)),
            pl.BlockSpec((1, H, 2 * I), lambda e, t: (e, 0, 0)),
            pl.BlockSpec((1, 1, 2 * I), lambda e, t: (e, 0, 0)),
        ],
        out_specs=pl.BlockSpec((1, BT, I), lambda e, t: (e, t, 0)),
        out_shape=jax.ShapeDtypeStruct((E, S, I), bf16),
    )(h2b, gw, gb)

    out = _call(
        _down_body,
        grid=(NT, E),
        in_specs=[
            pl.BlockSpec((1, BT, I), lambda t, e: (e, t, 0)),
            pl.BlockSpec((1, I, H), lambda t, e: (e, 0, 0)),
            pl.BlockSpec((1, 1, H), lambda t, e: (e, 0, 0)),
            pl.BlockSpec((BT, E), lambda t, e: (t, 0)),
            pl.BlockSpec((BT, H), lambda t, e: (t, 0)),
        ],
        out_specs=pl.BlockSpec((BT, H), lambda t, e: (t, 0)),
        out_shape=jax.ShapeDtypeStruct((S, H), f32),
        scratch_shapes=[pltpu.VMEM((BT, H), f32)],
    )(act, down_proj.astype(bf16), down_bias.reshape(E, 1, H), scores, res2)

    return out.reshape(B, S, H)


# speedup vs baseline: 5.7418x; 2.1018x over previous
import jax
import jax.numpy as jnp
from jax.experimental import pallas as pl
from jax.experimental.pallas import tpu as pltpu

_call = pl.pallas_call

B, S, H = 1, 2048, 1024
NH, KVH, HD = 16, 4, 64
E, I = 8, 1024
EPS = 1e-06
ALPHA = 1.702
LIMIT = 7.0
SCALING = HD ** -0.5
BT = 256
NT = S // BT
QKV = NH * HD + 2 * KVH * HD
RH = HD // 2


def _qkv_body(x_ref, ln_ref, w_ref, o_ref):
    x = x_ref[...]
    var = jnp.mean(x * x, axis=-1, keepdims=True)
    xn = (x * jax.lax.rsqrt(var + EPS) * ln_ref[...]).astype(jnp.bfloat16)
    o_ref[...] = jnp.dot(xn, w_ref[...], preferred_element_type=jnp.float32)


def _rope(x, c, s):
    x1 = x[:, :RH]
    x2 = x[:, RH:]
    return jnp.concatenate([x1 * c - x2 * s, x2 * c + x1 * s], axis=1)


def _attn_body(q_ref, k_ref, v_ref, cq_ref, sq_ref, ck_ref, sk_ref,
               sink_ref, o_ref):
    h = pl.program_id(0)
    qr = _rope(q_ref[0], cq_ref[...], sq_ref[...]).astype(jnp.bfloat16)
    kr = _rope(k_ref[0], ck_ref[...], sk_ref[...]).astype(jnp.bfloat16)
    scores = jax.lax.dot_general(
        qr, kr, (((1,), (1,)), ((), ())),
        preferred_element_type=jnp.float32) * SCALING
    sel = jax.lax.broadcasted_iota(jnp.int32, (1, NH), 1) == h
    snk = jnp.sum(jnp.where(sel, sink_ref[...], 0.0), axis=1, keepdims=True)
    m = jnp.maximum(jnp.max(scores, axis=1, keepdims=True), snk)
    p = jnp.exp(scores - m)
    denom = jnp.sum(p, axis=1, keepdims=True) + jnp.exp(snk - m)
    probs = (p / denom).astype(jnp.bfloat16)
    o_ref[0] = jnp.dot(probs, v_ref[0].astype(jnp.bfloat16),
                       preferred_element_type=jnp.float32)


def _proj_router_body(ao_ref, wo_ref, res_ref, ln2_ref, rw_ref, rb_ref,
                      res2_ref, h2_ref, sc_ref):
    attn = jnp.dot(ao_ref[...].astype(jnp.bfloat16), wo_ref[...],
                   preferred_element_type=jnp.float32)
    res2 = res_ref[...] + attn
    res2_ref[...] = res2
    var = jnp.mean(res2 * res2, axis=-1, keepdims=True)
    h2 = res2 * jax.lax.rsqrt(var + EPS) * ln2_ref[...]
    h2_ref[...] = h2
    rl = jnp.dot(h2, rw_ref[...],
                 preferred_element_type=jnp.float32) + rb_ref[...]
    v1 = jnp.max(rl, axis=1, keepdims=True)
    rl_m = jnp.where(rl == v1, -jnp.inf, rl)
    v2 = jnp.max(rl_m, axis=1, keepdims=True)
    w1 = 1.0 / (1.0 + jnp.exp(v2 - v1))
    w2 = 1.0 - w1
    sc_ref[...] = jnp.where(rl == v1, w1, jnp.where(rl == v2, w2, 0.0))


def _gateup_body(x_ref, w_ref, b_ref, o_ref):
    gu = jnp.dot(x_ref[...], w_ref[0],
                 preferred_element_type=jnp.float32) + b_ref[0]
    up_sh = pltpu.roll(gu, 2 * I - 1, 1)
    gate = jnp.minimum(gu, LIMIT)
    up = jnp.clip(up_sh, -LIMIT, LIMIT)
    glu = gate * jax.nn.sigmoid(gate * ALPHA)
    o_ref[0] = ((up + 1.0) * glu).astype(jnp.bfloat16)


def _down_body(a_ref, w_ref, b_ref, sc_ref, res2_ref, o_ref, acc_ref):
    e = pl.program_id(1)
    y = jnp.dot(a_ref[0], w_ref[0],
                preferred_element_type=jnp.float32) + b_ref[0]
    sel = jax.lax.broadcasted_iota(jnp.int32, (1, E), 1) == e
    w_tok = jnp.sum(jnp.where(sel, sc_ref[...], 0.0), axis=1, keepdims=True)
    contrib = w_tok * y

    @pl.when(e == 0)
    def _():
        acc_ref[...] = res2_ref[...] + contrib

    @pl.when(e != 0)
    def _():
        acc_ref[...] += contrib

    @pl.when(e == E - 1)
    def _():
        o_ref[...] = acc_ref[...]


def kernel(hidden_states, cos, sin, attention_mask, ln1_w, ln2_w, Wq, Wk, Wv,
           Wo, sinks, router_w, router_b, gate_up_proj, gate_up_bias,
           down_proj, down_bias):
    f32 = jnp.float32
    bf16 = jnp.bfloat16
    x = hidden_states.reshape(S, H)
    wqkv = jnp.concatenate([Wq, Wk, Wv], axis=0).T.astype(bf16)

    qkv = _call(
        _qkv_body,
        grid=(NT,),
        in_specs=[
            pl.BlockSpec((BT, H), lambda i: (i, 0)),
            pl.BlockSpec((1, H), lambda i: (0, 0)),
            pl.BlockSpec((H, QKV), lambda i: (0, 0)),
        ],
        out_specs=pl.BlockSpec((BT, QKV), lambda i: (i, 0)),
        out_shape=jax.ShapeDtypeStruct((S, QKV), f32),
    )(x, ln1_w.reshape(1, H), wqkv)

    q = qkv[:, :NH * HD].reshape(S, NH, HD).transpose(1, 0, 2)
    k = qkv[:, NH * HD:NH * HD + KVH * HD].reshape(S, KVH, HD).transpose(1, 0, 2)
    v = qkv[:, NH * HD + KVH * HD:].reshape(S, KVH, HD).transpose(1, 0, 2)
    cosf = cos.reshape(S, RH)
    sinf = sin.reshape(S, RH)

    ao = _call(
        _attn_body,
        grid=(NH, NT),
        in_specs=[
            pl.BlockSpec((1, BT, HD), lambda h, t: (h, t, 0)),
            pl.BlockSpec((1, S, HD), lambda h, t: (h // 4, 0, 0)),
            pl.BlockSpec((1, S, HD), lambda h, t: (h // 4, 0, 0)),
            pl.BlockSpec((BT, RH), lambda h, t: (t, 0)),
            pl.BlockSpec((BT, RH), lambda h, t: (t, 0)),
            pl.BlockSpec((S, RH), lambda h, t: (0, 0)),
            pl.BlockSpec((S, RH), lambda h, t: (0, 0)),
            pl.BlockSpec((1, NH), lambda h, t: (0, 0)),
        ],
        out_specs=pl.BlockSpec((1, BT, HD), lambda h, t: (h, t, 0)),
        out_shape=jax.ShapeDtypeStruct((NH, S, HD), f32),
    )(q, k, v, cosf, sinf, cosf, sinf, sinks.reshape(1, NH))

    aof = ao.transpose(1, 0, 2).reshape(S, NH * HD)

    res2, h2, scores = _call(
        _proj_router_body,
        grid=(NT,),
        in_specs=[
            pl.BlockSpec((BT, NH * HD), lambda i: (i, 0)),
            pl.BlockSpec((NH * HD, H), lambda i: (0, 0)),
            pl.BlockSpec((BT, H), lambda i: (i, 0)),
            pl.BlockSpec((1, H), lambda i: (0, 0)),
            pl.BlockSpec((H, E), lambda i: (0, 0)),
            pl.BlockSpec((1, E), lambda i: (0, 0)),
        ],
        out_specs=[
            pl.BlockSpec((BT, H), lambda i: (i, 0)),
            pl.BlockSpec((BT, H), lambda i: (i, 0)),
            pl.BlockSpec((BT, E), lambda i: (i, 0)),
        ],
        out_shape=[
            jax.ShapeDtypeStruct((S, H), f32),
            jax.ShapeDtypeStruct((S, H), f32),
            jax.ShapeDtypeStruct((S, E), f32),
        ],
    )(aof, Wo.T.astype(bf16), x, ln2_w.reshape(1, H),
      router_w.T.astype(f32), router_b.reshape(1, E))

    h2b = h2.astype(bf16)
    gw = gate_up_proj.astype(bf16)
    gb = gate_up_bias.reshape(E, 1, 2 * I)
    dwb = down_proj.astype(bf16)
    dw2 = jnp.stack([dwb, jnp.zeros_like(dwb)], axis=2).reshape(E, 2 * I, H)

    act = _call(
        _gateup_body,
        grid=(E, NT),
        in_specs=[
            pl.BlockSpec((BT, H), lambda e, t: (t, 0)),
            pl.BlockSpec((1, H, 2 * I), lambda e, t: (e, 0, 0)),
            pl.BlockSpec((1, 1, 2 * I), lambda e, t: (e, 0, 0)),
        ],
        out_specs=pl.BlockSpec((1, BT, 2 * I), lambda e, t: (e, t, 0)),
        out_shape=jax.ShapeDtypeStruct((E, S, 2 * I), bf16),
    )(h2b, gw, gb)

    out = _call(
        _down_body,
        grid=(NT, E),
        in_specs=[
            pl.BlockSpec((1, BT, 2 * I), lambda t, e: (e, t, 0)),
            pl.BlockSpec((1, 2 * I, H), lambda t, e: (e, 0, 0)),
            pl.BlockSpec((1, 1, H), lambda t, e: (e, 0, 0)),
            pl.BlockSpec((BT, E), lambda t, e: (t, 0)),
            pl.BlockSpec((BT, H), lambda t, e: (t, 0)),
        ],
        out_specs=pl.BlockSpec((BT, H), lambda t, e: (t, 0)),
        out_shape=jax.ShapeDtypeStruct((S, H), f32),
        scratch_shapes=[pltpu.VMEM((BT, H), f32)],
    )(act, dw2, down_bias.reshape(E, 1, H), scores, res2)

    return out.reshape(B, S, H)
